# baseline (speedup 1.0000x reference)
import functools

import numpy as np
import jax
import jax.numpy as jnp
from jax import lax
from jax.experimental import pallas as pl
from jax.experimental.pallas import tpu as pltpu
from jax.experimental.pallas import tpu_sc as plsc

B = 1024
C = 128
T = 256
NW = 32
RPW = B // NW

_SCALE = np.float32(2.5e-4)
_DT = np.float32(1e-6)
_MAGIC = np.float32(8388608.0)


def _body(cv_hbm, out_hbm, cv_v, buf_a, buf_b, buf_c, sem0, sem1, sem2):
    bufs = (buf_a, buf_b, buf_c)
    sems = (sem0, sem1, sem2)
    wid = lax.axis_index("s") * 2 + lax.axis_index("c")
    base = wid * RPW

    pltpu.sync_copy(cv_hbm.at[pl.ds(base, RPW)], cv_v)

    zeros16 = jnp.zeros((16,), jnp.float32)

    def _memset(buf):
        def step(i, carry):
            for j in range(C // 16):
                buf[i, pl.ds(j * 16, 16)] = zeros16
            return carry

        lax.fori_loop(0, T, step, 0)

    _memset(buf_a)

    ones16 = jnp.full((16,), 1.0, jnp.float32)
    lane = lax.iota(jnp.int32, 16)

    copies = [None, None, None]
    prev_idx = [None, None, None]

    for r in range(RPW):
        bsel = r % 3
        buf = bufs[bsel]

        if r == 1:
            _memset(buf_b)
        if r == 2:
            _memset(buf_c)
        if copies[bsel] is not None:
            copies[bsel].wait()
            for old_t, old_c in prev_idx[bsel]:
                plsc.store_scatter(buf, [old_t, old_c], zeros16)

        idxs = []
        for j in range(C // 16):
            cv16 = cv_v[r, pl.ds(j * 16, 16)]
            x = (cv16 * _SCALE) / _DT
            t = ((x + _MAGIC) - _MAGIC).astype(jnp.int32)
            c = (j * 16) + lane
            plsc.store_scatter(buf, [t, c], ones16)
            idxs.append((t, c))
        prev_idx[bsel] = idxs

        copies[bsel] = pltpu.async_copy(buf, out_hbm.at[base + r], sems[bsel])

    copies[0].wait()
    copies[1].wait()
    copies[2].wait()


@jax.jit
def kernel(coordinate_values):
    mesh = plsc.VectorSubcoreMesh(core_axis_name="c", subcore_axis_name="s")
    run = functools.partial(
        pl.kernel,
        out_type=jax.ShapeDtypeStruct((B, T, C), jnp.float32),
        mesh=mesh,
        compiler_params=pltpu.CompilerParams(needs_layout_passes=False),
        scratch_types=[
            pltpu.VMEM((RPW, C), jnp.float32),
            pltpu.VMEM((T, C), jnp.float32),
            pltpu.VMEM((T, C), jnp.float32),
            pltpu.VMEM((T, C), jnp.float32),
            pltpu.SemaphoreType.DMA,
            pltpu.SemaphoreType.DMA,
            pltpu.SemaphoreType.DMA,
        ],
    )(_body)
    return run(coordinate_values)


# speedup vs baseline: 1.0065x; 1.0065x over previous
import functools

import numpy as np
import jax
import jax.numpy as jnp
from jax import lax
from jax.experimental import pallas as pl
from jax.experimental.pallas import tpu as pltpu
from jax.experimental.pallas import tpu_sc as plsc

B = 1024
C = 128
T = 256
NW = 32
RPW = B // NW

_SCALE = np.float32(2.5e-4)
_DT = np.float32(1e-6)
_MAGIC = np.float32(8388608.0)


def _body(cv_hbm, out_hbm, cv_v, buf_a, buf_b, buf_c, sem0, sem1, sem2):
    bufs = (buf_a, buf_b, buf_c)
    sems = (sem0, sem1, sem2)
    wid = lax.axis_index("s") * 2 + lax.axis_index("c")
    base = wid * RPW

    pltpu.sync_copy(cv_hbm.at[pl.ds(base, RPW)], cv_v)

    zeros16 = jnp.zeros((16,), jnp.float32)

    def _memset(buf):
        def step(i, carry):
            for j in range(C // 16):
                buf[i, pl.ds(j * 16, 16)] = zeros16
            return carry

        lax.fori_loop(0, T, step, 0)

    _memset(buf_a)

    ones16 = jnp.full((16,), 1.0, jnp.float32)
    lane = lax.iota(jnp.int32, 16)

    copies = [None, None, None]
    prev_idx = [None, None, None]

    for r in range(RPW):
        bsel = r % 3
        buf = bufs[bsel]

        if r == 1:
            _memset(buf_b)
        if r == 2:
            _memset(buf_c)
        if copies[bsel] is not None:
            copies[bsel].wait()
            for old_t, old_c in prev_idx[bsel]:
                plsc.store_scatter(buf, [old_t, old_c], zeros16)

        idxs = []
        for j in range(C // 16):
            cv16 = cv_v[r, pl.ds(j * 16, 16)]
            x = (cv16 * _SCALE) / _DT
            t = ((x + _MAGIC) - _MAGIC).astype(jnp.int32)
            c = (j * 16) + lane
            plsc.store_scatter(buf, [t, c], ones16)
            idxs.append((t, c))
        prev_idx[bsel] = idxs

        copies[bsel] = pltpu.async_copy(buf, out_hbm.at[base + r], sems[bsel])

    copies[0].wait()
    copies[1].wait()
    copies[2].wait()


@jax.jit
def kernel(coordinate_values):
    mesh = plsc.VectorSubcoreMesh(core_axis_name="c", subcore_axis_name="s")
    run = functools.partial(
        pl.kernel,
        out_type=jax.ShapeDtypeStruct((B, T, C), jnp.float32),
        mesh=mesh,
        compiler_params=pltpu.CompilerParams(
            needs_layout_passes=False, skip_device_barrier=True
        ),
        scratch_types=[
            pltpu.VMEM((RPW, C), jnp.float32),
            pltpu.VMEM((T, C), jnp.float32),
            pltpu.VMEM((T, C), jnp.float32),
            pltpu.VMEM((T, C), jnp.float32),
            pltpu.SemaphoreType.DMA,
            pltpu.SemaphoreType.DMA,
            pltpu.SemaphoreType.DMA,
        ],
    )(_body)
    return run(coordinate_values)
